# baseline (speedup 1.0000x reference)
import functools
import math

import jax
import jax.numpy as jnp
from jax import lax
from jax.experimental import pallas as pl
from jax.experimental.pallas import tpu as pltpu
from jax.experimental.pallas import tpu_sc as plsc

VOCAB = 1000000
D = 64
BATCH = 16384
HIST = 200
B = BATCH * HIST

NC, NS, L = 2, 16, 16
NW = NC * NS
PER_W = B // NW
CHUNK = 128
NSTEPS = PER_W // CHUNK

SCALE = math.sqrt(D)


def _make_sc_gather():
  mesh = plsc.VectorSubcoreMesh(
      core_axis_name="c", subcore_axis_name="s", num_cores=NC
  )

  @functools.partial(
      pl.kernel,
      mesh=mesh,
      out_type=jax.ShapeDtypeStruct((B, D), jnp.float32),
      compiler_params=pltpu.CompilerParams(use_tc_tiling_on_sc=False),
      scratch_types=[
          pltpu.VMEM((CHUNK,), jnp.int32),
          pltpu.VMEM((CHUNK, D), jnp.float32),
          pltpu.SemaphoreType.DMA,
      ],
  )
  def sc_gather(idx_hbm, table_hbm, out_hbm, idx_v, rows_v, sem):
    wid = lax.axis_index("s") * NC + lax.axis_index("c")
    wbase = wid * PER_W

    def step(i, carry):
      base = wbase + i * CHUNK
      pltpu.sync_copy(idx_hbm.at[pl.ds(base, CHUNK)], idx_v)
      pltpu.async_copy(table_hbm.at[idx_v], rows_v, sem).wait()

      def scale_row(r, c):
        for j in range(D // L):
          rows_v[r, pl.ds(j * L, L)] = rows_v[r, pl.ds(j * L, L)] * SCALE
        return c

      lax.fori_loop(0, CHUNK, scale_row, 0)
      pltpu.sync_copy(rows_v, out_hbm.at[pl.ds(base, CHUNK)])
      return carry

    lax.fori_loop(0, NSTEPS, step, 0)

  return sc_gather


def kernel(x, table):
  idx = x.reshape(-1).astype(jnp.int32)
  out = _make_sc_gather()(idx, table)
  return out.reshape(BATCH, HIST, D)


# speedup vs baseline: 1.2589x; 1.2589x over previous
import functools
import math

import jax
import jax.numpy as jnp
from jax import lax
from jax.experimental import pallas as pl
from jax.experimental.pallas import tpu as pltpu
from jax.experimental.pallas import tpu_sc as plsc

VOCAB = 1000000
D = 64
BATCH = 16384
HIST = 200
B = BATCH * HIST

NC, NS = 2, 16
NW = NC * NS
PER_W = B // NW

GL = 128
GPB = 2
RB = GL * GPB
NB = PER_W // RB
NBUF = 4
NT = NB // NBUF

SCALE = math.sqrt(D)

TC_ROWS = 8000


def _tc_prescale(table):
  def body(t_ref, o_ref):
    o_ref[...] = t_ref[...] * SCALE

  return pl.pallas_call(
      body,
      grid=(VOCAB // TC_ROWS,),
      in_specs=[pl.BlockSpec((TC_ROWS, D), lambda i: (i, 0))],
      out_specs=pl.BlockSpec((TC_ROWS, D), lambda i: (i, 0)),
      out_shape=jax.ShapeDtypeStruct((VOCAB, D), jnp.float32),
  )(table)


def _make_sc_gather():
  mesh = plsc.VectorSubcoreMesh(
      core_axis_name="c", subcore_axis_name="s", num_cores=NC
  )

  scratch = (
      [pltpu.VMEM((NBUF, GPB, GL), jnp.int32)]
      + [pltpu.VMEM((GPB, GL, D), jnp.float32)] * NBUF
      + [pltpu.SemaphoreType.DMA] * (3 * NBUF)
  )

  @functools.partial(
      pl.kernel,
      mesh=mesh,
      out_type=jax.ShapeDtypeStruct((B // GL, GL, D), jnp.float32),
      compiler_params=pltpu.CompilerParams(use_tc_tiling_on_sc=False),
      scratch_types=scratch,
  )
  def sc_gather(idx_hbm, table_hbm, out_hbm, idx4, r0, r1, r2, r3, *sems):
    rows = [r0, r1, r2, r3]
    gsem = sems[0:NBUF]
    wsem = sems[NBUF : 2 * NBUF]
    isem = sems[2 * NBUF : 3 * NBUF]

    wid = lax.axis_index("s") * NC + lax.axis_index("c")
    wblk = wid * NB

    def idx_sync(g, b):
      pltpu.sync_copy(idx_hbm.at[wblk + g], idx4.at[b])

    def idx_fire(g, b):
      pltpu.async_copy(idx_hbm.at[wblk + g], idx4.at[b], isem[b])

    def idx_wait(g, b):
      pltpu.make_async_copy(idx_hbm.at[wblk + g], idx4.at[b], isem[b]).wait()

    def gather_fire(b):
      for j in range(GPB):
        pltpu.async_copy(table_hbm.at[idx4.at[b, j]], rows[b].at[j], gsem[b])

    def gather_wait(b):
      for j in range(GPB):
        pltpu.make_async_copy(
            table_hbm.at[idx4.at[b, j]], rows[b].at[j], gsem[b]
        ).wait()

    def wb_fire(g, b):
      pltpu.async_copy(
          rows[b], out_hbm.at[pl.ds((wblk + g) * GPB, GPB)], wsem[b]
      )

    def wb_wait(g, b):
      pltpu.make_async_copy(
          rows[b], out_hbm.at[pl.ds((wblk + g) * GPB, GPB)], wsem[b]
      ).wait()

    idx_sync(0, 0)
    idx_sync(1, 1)
    idx_sync(2, 2)
    gather_fire(0)
    gather_fire(1)

    def step(g, k, *, skip_isem_wait=False, fire_idx=True, refill=True,
             wait_wb=True):
      gather_wait(k)
      wb_fire(g, k)
      if refill:
        b2 = (k + 2) % NBUF
        if wait_wb:
          wb_wait(g - 2, b2)
        if not skip_isem_wait:
          idx_wait(g + 2, b2)
        gather_fire(b2)
        if fire_idx:
          idx_fire(g + 3, (k + 3) % NBUF)

    step(0, 0, skip_isem_wait=True, wait_wb=False)
    step(1, 1, wait_wb=False)
    step(2, 2)
    step(3, 3)

    def group(t, c):
      for k in range(NBUF):
        step(t * NBUF + k, k)
      return c

    lax.fori_loop(1, NT - 1, group, 0)

    g0 = NB - NBUF
    step(g0 + 0, 0)
    step(g0 + 1, 1, fire_idx=False)
    step(g0 + 2, 2, refill=False)
    step(g0 + 3, 3, refill=False)

    for k in range(NBUF):
      wb_wait(g0 + k, k)

  return sc_gather


def kernel(x, table):
  idx = x.reshape(-1).astype(jnp.int32).reshape(B // GL // GPB, GPB, GL)
  scaled = _tc_prescale(table)
  out = _make_sc_gather()(idx, scaled)
  return out.reshape(BATCH, HIST, D)


# speedup vs baseline: 1.4311x; 1.1368x over previous
import functools
import math

import jax
import jax.numpy as jnp
from jax import lax
from jax.experimental import pallas as pl
from jax.experimental.pallas import tpu as pltpu
from jax.experimental.pallas import tpu_sc as plsc

VOCAB = 1000000
D = 64
BATCH = 16384
HIST = 200

NC, NS, L = 2, 16, 16
NW = NC * NS
NB = BATCH // NW
NBUF = 4
NT = NB // NBUF

GATHER_SPLITS = ((0, 128), (128, 72))

SCALE = math.sqrt(D)

ROWS_PER_ITER = 20


def _make_sc_kernel():
  mesh = plsc.VectorSubcoreMesh(
      core_axis_name="c", subcore_axis_name="s", num_cores=NC
  )

  scratch = (
      [pltpu.VMEM((NBUF, HIST), jnp.int32)]
      + [pltpu.VMEM((HIST, D), jnp.float32)] * NBUF
      + [pltpu.SemaphoreType.DMA] * (3 * NBUF)
  )

  @functools.partial(
      pl.kernel,
      mesh=mesh,
      out_type=jax.ShapeDtypeStruct((BATCH, HIST, D), jnp.float32),
      compiler_params=pltpu.CompilerParams(use_tc_tiling_on_sc=False),
      scratch_types=scratch,
  )
  def sc_kernel(idx_hbm, table_hbm, out_hbm, idx4, r0, r1, r2, r3, *sems):
    rows = [r0, r1, r2, r3]
    gsem = sems[0:NBUF]
    wsem = sems[NBUF : 2 * NBUF]
    isem = sems[2 * NBUF : 3 * NBUF]

    wid = lax.axis_index("s") * NC + lax.axis_index("c")
    wblk = wid * NB

    def idx_sync(g, b):
      pltpu.sync_copy(idx_hbm.at[wblk + g], idx4.at[b])

    def idx_fire(g, b):
      pltpu.async_copy(idx_hbm.at[wblk + g], idx4.at[b], isem[b])

    def idx_wait(g, b):
      pltpu.make_async_copy(idx_hbm.at[wblk + g], idx4.at[b], isem[b]).wait()

    def gather_fire(b):
      for off, sz in GATHER_SPLITS:
        pltpu.async_copy(
            table_hbm.at[idx4.at[b, pl.ds(off, sz)]],
            rows[b].at[pl.ds(off, sz)],
            gsem[b],
        )

    def gather_wait(b):
      for off, sz in GATHER_SPLITS:
        pltpu.make_async_copy(
            table_hbm.at[idx4.at[b, pl.ds(off, sz)]],
            rows[b].at[pl.ds(off, sz)],
            gsem[b],
        ).wait()

    def wb_fire(g, b):
      pltpu.async_copy(rows[b], out_hbm.at[wblk + g], wsem[b])

    def wb_wait(g, b):
      pltpu.make_async_copy(rows[b], out_hbm.at[wblk + g], wsem[b]).wait()

    def scale_buf(b):
      def body(r, c):
        base = r * ROWS_PER_ITER
        for i in range(ROWS_PER_ITER):
          for c4 in range(D // L):
            sl = pl.ds(c4 * L, L)
            rows[b][base + i, sl] = rows[b][base + i, sl] * SCALE
        return c

      lax.fori_loop(0, HIST // ROWS_PER_ITER, body, 0)

    idx_sync(0, 0)
    idx_sync(1, 1)
    idx_sync(2, 2)
    gather_fire(0)
    gather_fire(1)

    def step(g, k, *, skip_isem_wait=False, fire_idx=True, refill=True,
             wait_wb=True):
      gather_wait(k)
      scale_buf(k)
      wb_fire(g, k)
      if refill:
        b2 = (k + 2) % NBUF
        if wait_wb:
          wb_wait(g - 2, b2)
        if not skip_isem_wait:
          idx_wait(g + 2, b2)
        gather_fire(b2)
        if fire_idx:
          idx_fire(g + 3, (k + 3) % NBUF)

    step(0, 0, skip_isem_wait=True, wait_wb=False)
    step(1, 1, wait_wb=False)
    step(2, 2)
    step(3, 3)

    def group(t, c):
      for k in range(NBUF):
        step(t * NBUF + k, k)
      return c

    lax.fori_loop(1, NT - 1, group, 0)

    g0 = NB - NBUF
    step(g0 + 0, 0)
    step(g0 + 1, 1, fire_idx=False)
    step(g0 + 2, 2, refill=False)
    step(g0 + 3, 3, refill=False)

    for k in range(NBUF):
      wb_wait(g0 + k, k)

  return sc_kernel


def kernel(x, table):
  return _make_sc_kernel()(x.astype(jnp.int32), table)
